# baseline (speedup 1.0000x reference)
import functools

import jax
import jax.numpy as jnp
from jax.experimental import pallas as pl
from jax.experimental.pallas import tpu as pltpu

_B, _N, _D = 8, 4096, 768
_C = 16
_NUM_CLASSES = 2
_THR = 0.8
_BN = 512


def _rdd_body(c_ref, x_ref, w_ref, b_ref, o_ref, sums, counts):
    n = pl.program_id(1)
    nb = pl.num_programs(1)

    @pl.when(n == 0)
    def _zero():
        sums[...] = jnp.zeros_like(sums)
        counts[...] = jnp.zeros_like(counts)

    x = x_ref[0]
    cid = jax.lax.broadcasted_iota(jnp.int32, (_C, _BN), 0)
    mask = (c_ref[0] == cid).astype(jnp.float32)
    sums[...] += jnp.dot(mask, x, preferred_element_type=jnp.float32)
    counts[...] += jnp.sum(mask, axis=1, keepdims=True)

    @pl.when(n == nb - 1)
    def _finish():
        bidx = pl.program_id(0)
        feats = sums[...] / jnp.maximum(counts[...], 1.0)
        logits = (
            jnp.dot(feats, w_ref[...], preferred_element_type=jnp.float32)
            + b_ref[0]
        )
        d = logits[:, 1:2] - logits[:, 0:1]
        dmax = jnp.max(d)
        dmin = jnp.min(d)
        use_min = jax.nn.sigmoid(dmax) < _THR
        target = jnp.where(use_min, dmin, dmax)
        idxs = jax.lax.broadcasted_iota(jnp.int32, (_C, 1), 0)
        sel = jnp.min(jnp.where(d == target, idxs, _C))
        selmask = (idxs == sel).astype(jnp.float32)
        out = jnp.sum(selmask * logits, axis=0, keepdims=True)
        o_ref[pl.ds(bidx, 1), :] = out


@functools.partial(jax.jit, static_argnames=())
def _run(inst_feat, clusters, W, b2):
    grid = (_B, _N // _BN)
    return pl.pallas_call(
        _rdd_body,
        grid=grid,
        in_specs=[
            pl.BlockSpec((1, 1, _BN), lambda i, j: (i, 0, j)),
            pl.BlockSpec((1, _BN, _D), lambda i, j: (i, j, 0)),
            pl.BlockSpec((_D, _NUM_CLASSES), lambda i, j: (0, 0)),
            pl.BlockSpec((1, _NUM_CLASSES), lambda i, j: (0, 0)),
        ],
        out_specs=pl.BlockSpec((_B, _NUM_CLASSES), lambda i, j: (0, 0)),
        out_shape=jax.ShapeDtypeStruct((_B, _NUM_CLASSES), jnp.float32),
        scratch_shapes=[
            pltpu.VMEM((_C, _D), jnp.float32),
            pltpu.VMEM((_C, 1), jnp.float32),
        ],
    )(clusters, inst_feat, W, b2)


def kernel(inst_feat, clusters_idcs, W, b):
    clusters = clusters_idcs.astype(jnp.int32).reshape(_B, 1, _N)
    b2 = b.reshape(1, _NUM_CLASSES).astype(jnp.float32)
    return _run(inst_feat, clusters, W, b2)


# speedup vs baseline: 1.2375x; 1.2375x over previous
import functools

import jax
import jax.numpy as jnp
from jax.experimental import pallas as pl
from jax.experimental.pallas import tpu as pltpu

_B, _N, _D = 8, 4096, 768
_C = 16
_NUM_CLASSES = 2
_THR = 0.8
_BN = 1024


def _rdd_body(c_ref, x_ref, w_ref, b_ref, o_ref, sums, counts):
    n = pl.program_id(1)
    nb = pl.num_programs(1)

    @pl.when(n == 0)
    def _zero():
        sums[...] = jnp.zeros_like(sums)
        counts[...] = jnp.zeros_like(counts)

    x = x_ref[0]
    cid = jax.lax.broadcasted_iota(jnp.int32, (_C, _BN), 0)
    mask = (c_ref[0] == cid).astype(jnp.bfloat16)
    x_hi = x.astype(jnp.bfloat16)
    x_lo = (x - x_hi.astype(jnp.float32)).astype(jnp.bfloat16)
    sums[...] += jnp.dot(
        mask, x_hi, preferred_element_type=jnp.float32
    ) + jnp.dot(mask, x_lo, preferred_element_type=jnp.float32)
    counts[...] += jnp.sum(
        mask.astype(jnp.float32), axis=1, keepdims=True
    )

    @pl.when(n == nb - 1)
    def _finish():
        bidx = pl.program_id(0)
        feats = sums[...] / jnp.maximum(counts[...], 1.0)
        logits = (
            jnp.dot(feats, w_ref[...], preferred_element_type=jnp.float32)
            + b_ref[0]
        )
        d = logits[:, 1:2] - logits[:, 0:1]
        dmax = jnp.max(d)
        dmin = jnp.min(d)
        use_min = jax.nn.sigmoid(dmax) < _THR
        target = jnp.where(use_min, dmin, dmax)
        idxs = jax.lax.broadcasted_iota(jnp.int32, (_C, 1), 0)
        sel = jnp.min(jnp.where(d == target, idxs, _C))
        selmask = (idxs == sel).astype(jnp.float32)
        out = jnp.sum(selmask * logits, axis=0, keepdims=True)
        o_ref[pl.ds(bidx, 1), :] = out


@functools.partial(jax.jit, static_argnames=())
def _run(inst_feat, clusters, W, b2):
    grid = (_B, _N // _BN)
    return pl.pallas_call(
        _rdd_body,
        grid=grid,
        in_specs=[
            pl.BlockSpec((1, 1, _BN), lambda i, j: (i, 0, j)),
            pl.BlockSpec((1, _BN, _D), lambda i, j: (i, j, 0)),
            pl.BlockSpec((_D, _NUM_CLASSES), lambda i, j: (0, 0)),
            pl.BlockSpec((1, _NUM_CLASSES), lambda i, j: (0, 0)),
        ],
        out_specs=pl.BlockSpec((_B, _NUM_CLASSES), lambda i, j: (0, 0)),
        out_shape=jax.ShapeDtypeStruct((_B, _NUM_CLASSES), jnp.float32),
        scratch_shapes=[
            pltpu.VMEM((_C, _D), jnp.float32),
            pltpu.VMEM((_C, 1), jnp.float32),
        ],
    )(clusters, inst_feat, W, b2)


def kernel(inst_feat, clusters_idcs, W, b):
    clusters = clusters_idcs.astype(jnp.int32).reshape(_B, 1, _N)
    b2 = b.reshape(1, _NUM_CLASSES).astype(jnp.float32)
    return _run(inst_feat, clusters, W, b2)


# speedup vs baseline: 1.5924x; 1.2868x over previous
import functools

import jax
import jax.numpy as jnp
from jax.experimental import pallas as pl
from jax.experimental.pallas import tpu as pltpu

_B, _N, _D = 8, 4096, 768
_C = 16
_NUM_CLASSES = 2
_THR = 0.8
_BN = 2048


def _rdd_body(c_ref, x_ref, w_ref, b_ref, o_ref, sums, counts):
    n = pl.program_id(1)
    nb = pl.num_programs(1)

    @pl.when(n == 0)
    def _zero():
        sums[...] = jnp.zeros_like(sums)
        counts[...] = jnp.zeros_like(counts)

    x = x_ref[0]
    cid = jax.lax.broadcasted_iota(jnp.int32, (_C, _BN), 0)
    mask = (c_ref[0] == cid).astype(jnp.bfloat16)
    x_hi = x.astype(jnp.bfloat16)
    x_lo = (x - x_hi.astype(jnp.float32)).astype(jnp.bfloat16)
    sums[...] += jnp.dot(
        mask, x_hi, preferred_element_type=jnp.float32
    ) + jnp.dot(mask, x_lo, preferred_element_type=jnp.float32)
    counts[...] += jnp.sum(
        mask.astype(jnp.float32), axis=1, keepdims=True
    )

    @pl.when(n == nb - 1)
    def _finish():
        bidx = pl.program_id(0)
        feats = sums[...] / jnp.maximum(counts[...], 1.0)
        logits = (
            jnp.dot(feats, w_ref[...], preferred_element_type=jnp.float32)
            + b_ref[0]
        )
        d = logits[:, 1:2] - logits[:, 0:1]
        dmax = jnp.max(d)
        dmin = jnp.min(d)
        use_min = jax.nn.sigmoid(dmax) < _THR
        target = jnp.where(use_min, dmin, dmax)
        idxs = jax.lax.broadcasted_iota(jnp.int32, (_C, 1), 0)
        sel = jnp.min(jnp.where(d == target, idxs, _C))
        selmask = (idxs == sel).astype(jnp.float32)
        out = jnp.sum(selmask * logits, axis=0, keepdims=True)
        o_ref[pl.ds(bidx, 1), :] = out


@functools.partial(jax.jit, static_argnames=())
def _run(inst_feat, clusters, W, b2):
    grid = (_B, _N // _BN)
    return pl.pallas_call(
        _rdd_body,
        grid=grid,
        in_specs=[
            pl.BlockSpec((1, 1, _BN), lambda i, j: (i, 0, j)),
            pl.BlockSpec((1, _BN, _D), lambda i, j: (i, j, 0)),
            pl.BlockSpec((_D, _NUM_CLASSES), lambda i, j: (0, 0)),
            pl.BlockSpec((1, _NUM_CLASSES), lambda i, j: (0, 0)),
        ],
        out_specs=pl.BlockSpec((_B, _NUM_CLASSES), lambda i, j: (0, 0)),
        out_shape=jax.ShapeDtypeStruct((_B, _NUM_CLASSES), jnp.float32),
        scratch_shapes=[
            pltpu.VMEM((_C, _D), jnp.float32),
            pltpu.VMEM((_C, 1), jnp.float32),
        ],
    )(clusters, inst_feat, W, b2)


def kernel(inst_feat, clusters_idcs, W, b):
    clusters = clusters_idcs.astype(jnp.int32).reshape(_B, 1, _N)
    b2 = b.reshape(1, _NUM_CLASSES).astype(jnp.float32)
    return _run(inst_feat, clusters, W, b2)


# speedup vs baseline: 1.7115x; 1.0747x over previous
import functools

import jax
import jax.numpy as jnp
from jax.experimental import pallas as pl
from jax.experimental.pallas import tpu as pltpu

_B, _N, _D = 8, 4096, 768
_C = 16
_NUM_CLASSES = 2
_THR = 0.8
_BN = 4096


def _rdd_body(c_ref, x_ref, w_ref, b_ref, o_ref, sums, counts):
    n = pl.program_id(1)
    nb = pl.num_programs(1)

    @pl.when(n == 0)
    def _zero():
        sums[...] = jnp.zeros_like(sums)
        counts[...] = jnp.zeros_like(counts)

    x = x_ref[0]
    cid = jax.lax.broadcasted_iota(jnp.int32, (_C, _BN), 0)
    mask = (c_ref[0] == cid).astype(jnp.bfloat16)
    x_hi = x.astype(jnp.bfloat16)
    x_lo = (x - x_hi.astype(jnp.float32)).astype(jnp.bfloat16)
    sums[...] += jnp.dot(
        mask, x_hi, preferred_element_type=jnp.float32
    ) + jnp.dot(mask, x_lo, preferred_element_type=jnp.float32)
    counts[...] += jnp.sum(
        mask.astype(jnp.float32), axis=1, keepdims=True
    )

    @pl.when(n == nb - 1)
    def _finish():
        bidx = pl.program_id(0)
        feats = sums[...] / jnp.maximum(counts[...], 1.0)
        logits = (
            jnp.dot(feats, w_ref[...], preferred_element_type=jnp.float32)
            + b_ref[0]
        )
        d = logits[:, 1:2] - logits[:, 0:1]
        dmax = jnp.max(d)
        dmin = jnp.min(d)
        use_min = jax.nn.sigmoid(dmax) < _THR
        target = jnp.where(use_min, dmin, dmax)
        idxs = jax.lax.broadcasted_iota(jnp.int32, (_C, 1), 0)
        sel = jnp.min(jnp.where(d == target, idxs, _C))
        selmask = (idxs == sel).astype(jnp.float32)
        out = jnp.sum(selmask * logits, axis=0, keepdims=True)
        o_ref[pl.ds(bidx, 1), :] = out


@functools.partial(jax.jit, static_argnames=())
def _run(inst_feat, clusters, W, b2):
    grid = (_B, _N // _BN)
    return pl.pallas_call(
        _rdd_body,
        grid=grid,
        in_specs=[
            pl.BlockSpec((1, 1, _BN), lambda i, j: (i, 0, j)),
            pl.BlockSpec((1, _BN, _D), lambda i, j: (i, j, 0)),
            pl.BlockSpec((_D, _NUM_CLASSES), lambda i, j: (0, 0)),
            pl.BlockSpec((1, _NUM_CLASSES), lambda i, j: (0, 0)),
        ],
        out_specs=pl.BlockSpec((_B, _NUM_CLASSES), lambda i, j: (0, 0)),
        out_shape=jax.ShapeDtypeStruct((_B, _NUM_CLASSES), jnp.float32),
        scratch_shapes=[
            pltpu.VMEM((_C, _D), jnp.float32),
            pltpu.VMEM((_C, 1), jnp.float32),
        ],
    )(clusters, inst_feat, W, b2)


def kernel(inst_feat, clusters_idcs, W, b):
    clusters = clusters_idcs.astype(jnp.int32).reshape(_B, 1, _N)
    b2 = b.reshape(1, _NUM_CLASSES).astype(jnp.float32)
    return _run(inst_feat, clusters, W, b2)
